# Initial kernel scaffold; baseline (speedup 1.0000x reference)
import jax
import jax.numpy as jnp
from jax.experimental import pallas as pl


def kernel(lut, inputs, targets, epoch):
    raise NotImplementedError("write your pallas kernel here")


# speedup vs baseline: 2.3631x; 2.3631x over previous
import jax
import jax.numpy as jnp
from jax.experimental import pallas as pl
from jax.experimental.pallas import tpu as pltpu

_K = 5532
_C = 256
_KB = 512
_NBLK = 11
_NPIX = 2048
_N_TOT = 4096


def _oim_kernel(lut_ref, x_ref, tgt_ref, out_ref, m_ref, s_ref, tl_ref):
    b = pl.program_id(0)
    j = pl.program_id(1)

    @pl.when(j == 0)
    def _init():
        m_ref[...] = jnp.full((1, _NPIX), -1e30, jnp.float32)
        s_ref[...] = jnp.zeros((1, _NPIX), jnp.float32)
        tl_ref[...] = jnp.zeros((1, _NPIX), jnp.float32)

    x = x_ref[0]
    w = lut_ref[...]
    s_blk = jax.lax.dot_general(w, x, (((1,), (0,)), ((), ())),
                                preferred_element_type=jnp.float32)
    row = jax.lax.broadcasted_iota(jnp.int32, (_KB, _NPIX), 0) + j * _KB
    s_blk = jnp.where(row < _K, s_blk, -1e30)

    bm = jnp.max(s_blk, axis=0, keepdims=True)
    m_old = m_ref[...]
    m_new = jnp.maximum(m_old, bm)
    p = jnp.exp(s_blk - m_new)
    s_ref[...] = s_ref[...] * jnp.exp(m_old - m_new) + jnp.sum(p, axis=0, keepdims=True)
    m_ref[...] = m_new

    eq = row == tgt_ref[0]
    tl_ref[...] += jnp.sum(jnp.where(eq, s_blk, 0.0), axis=0, keepdims=True)

    @pl.when(j == _NBLK - 1)
    def _fin():
        nll = m_ref[...] + jnp.log(s_ref[...]) - tl_ref[...]
        part = jnp.sum(nll, axis=1, keepdims=True) * (1.0 / _N_TOT)

        @pl.when(b == 0)
        def _w():
            out_ref[...] = part

        @pl.when(b > 0)
        def _a():
            out_ref[...] += part


def kernel(lut, inputs, targets, epoch):
    lut_p = jnp.pad(lut, ((0, _NBLK * _KB - _K), (0, 0)))
    x = inputs.reshape(2, _C, _NPIX)
    tgt = targets.reshape(2, 1, _NPIX)
    out = pl.pallas_call(
        _oim_kernel,
        grid=(2, _NBLK),
        in_specs=[
            pl.BlockSpec((_KB, _C), lambda b, j: (j, 0)),
            pl.BlockSpec((1, _C, _NPIX), lambda b, j: (b, 0, 0)),
            pl.BlockSpec((1, 1, _NPIX), lambda b, j: (b, 0, 0)),
        ],
        out_specs=pl.BlockSpec((1, 1), lambda b, j: (0, 0)),
        out_shape=jax.ShapeDtypeStruct((1, 1), jnp.float32),
        scratch_shapes=[
            pltpu.VMEM((1, _NPIX), jnp.float32),
            pltpu.VMEM((1, _NPIX), jnp.float32),
            pltpu.VMEM((1, _NPIX), jnp.float32),
        ],
        compiler_params=pltpu.CompilerParams(
            dimension_semantics=("arbitrary", "arbitrary"),
        ),
    )(lut_p, x, tgt)
    loss = out[0, 0]
    return jnp.where(epoch < 0, jnp.float32(0.0), loss)


# speedup vs baseline: 3.3062x; 1.3991x over previous
import jax
import jax.numpy as jnp
from jax.experimental import pallas as pl
from jax.experimental.pallas import tpu as pltpu

_K = 5532
_C = 256
_KB = 512
_NBLK = 11
_NPIX = 2048
_N_TOT = 4096
_NPAD = _NBLK * _KB - _K


def _oim_kernel(lut_ref, x_ref, tgt_ref, out_ref, s_ref, tl_ref):
    b = pl.program_id(0)
    j = pl.program_id(1)

    @pl.when(j == 0)
    def _init():
        s_ref[...] = jnp.zeros((1, _NPIX), jnp.float32)
        tl_ref[...] = jnp.zeros((1, _NPIX), jnp.float32)

    x = x_ref[0]
    w = lut_ref[...]
    s_blk = jax.lax.dot_general(w, x, (((1,), (0,)), ((), ())),
                                preferred_element_type=jnp.float32)

    s_ref[...] += jnp.sum(jnp.exp(s_blk), axis=0, keepdims=True)

    row = jax.lax.broadcasted_iota(jnp.int32, (_KB, _NPIX), 0) + j * _KB
    eq = row == tgt_ref[0]
    tl_ref[...] += jnp.sum(jnp.where(eq, s_blk, 0.0), axis=0, keepdims=True)

    @pl.when(j == _NBLK - 1)
    def _fin():
        nll = jnp.log(s_ref[...] - _NPAD) - tl_ref[...]
        part = jnp.sum(nll, axis=1, keepdims=True) * (1.0 / _N_TOT)

        @pl.when(b == 0)
        def _w():
            out_ref[...] = part

        @pl.when(b > 0)
        def _a():
            out_ref[...] += part


def kernel(lut, inputs, targets, epoch):
    lut_p = jnp.pad(lut, ((0, _NBLK * _KB - _K), (0, 0)))
    x = inputs.reshape(2, _C, _NPIX)
    tgt = targets.reshape(2, 1, _NPIX)
    out = pl.pallas_call(
        _oim_kernel,
        grid=(2, _NBLK),
        in_specs=[
            pl.BlockSpec((_KB, _C), lambda b, j: (j, 0)),
            pl.BlockSpec((1, _C, _NPIX), lambda b, j: (b, 0, 0)),
            pl.BlockSpec((1, 1, _NPIX), lambda b, j: (b, 0, 0)),
        ],
        out_specs=pl.BlockSpec((1, 1), lambda b, j: (0, 0)),
        out_shape=jax.ShapeDtypeStruct((1, 1), jnp.float32),
        scratch_shapes=[
            pltpu.VMEM((1, _NPIX), jnp.float32),
            pltpu.VMEM((1, _NPIX), jnp.float32),
        ],
        compiler_params=pltpu.CompilerParams(
            dimension_semantics=("arbitrary", "arbitrary"),
        ),
    )(lut_p, x, tgt)
    loss = out[0, 0]
    return jnp.where(epoch < 0, jnp.float32(0.0), loss)


# speedup vs baseline: 3.3256x; 1.0059x over previous
import jax
import jax.numpy as jnp
from jax.experimental import pallas as pl
from jax.experimental.pallas import tpu as pltpu

_K = 5532
_C = 256
_KB = 512
_NBLK = 11
_NPIX = 2048
_N_TOT = 4096
_NPAD = _NBLK * _KB - _K


def _oim_kernel(lut_ref, x_ref, tgt_ref, out_ref, s_ref, tl_ref):
    b = pl.program_id(0)
    j = pl.program_id(1)

    @pl.when(j == 0)
    def _init():
        s_ref[...] = jnp.zeros((1, _NPIX), jnp.float32)
        tl_ref[...] = jnp.zeros((1, _NPIX), jnp.float32)

    x = x_ref[0]
    w = lut_ref[...]
    s_blk = jax.lax.dot_general(w, x, (((1,), (0,)), ((), ())),
                                preferred_element_type=jnp.float32)

    s_ref[...] += jnp.sum(jnp.exp(s_blk), axis=0, keepdims=True)

    row = jax.lax.broadcasted_iota(jnp.int32, (_KB, _NPIX), 0) + j * _KB
    eq = row == tgt_ref[0]
    tl_ref[...] += jnp.sum(jnp.where(eq, s_blk, 0.0), axis=0, keepdims=True)

    @pl.when(j == _NBLK - 1)
    def _fin():
        nll = jnp.log(s_ref[...] - _NPAD) - tl_ref[...]
        part = jnp.sum(nll, axis=1, keepdims=True) * (1.0 / _N_TOT)

        @pl.when(b == 0)
        def _w():
            out_ref[...] = part

        @pl.when(b > 0)
        def _a():
            out_ref[...] += part


def kernel(lut, inputs, targets, epoch):
    lut_p = jnp.pad(lut.astype(jnp.bfloat16), ((0, _NBLK * _KB - _K), (0, 0)))
    x = inputs.reshape(2, _C, _NPIX).astype(jnp.bfloat16)
    tgt = targets.reshape(2, 1, _NPIX)
    out = pl.pallas_call(
        _oim_kernel,
        grid=(2, _NBLK),
        in_specs=[
            pl.BlockSpec((_KB, _C), lambda b, j: (j, 0)),
            pl.BlockSpec((1, _C, _NPIX), lambda b, j: (b, 0, 0)),
            pl.BlockSpec((1, 1, _NPIX), lambda b, j: (b, 0, 0)),
        ],
        out_specs=pl.BlockSpec((1, 1), lambda b, j: (0, 0)),
        out_shape=jax.ShapeDtypeStruct((1, 1), jnp.float32),
        scratch_shapes=[
            pltpu.VMEM((1, _NPIX), jnp.float32),
            pltpu.VMEM((1, _NPIX), jnp.float32),
        ],
        compiler_params=pltpu.CompilerParams(
            dimension_semantics=("arbitrary", "arbitrary"),
        ),
    )(lut_p, x, tgt)
    loss = out[0, 0]
    return jnp.where(epoch < 0, jnp.float32(0.0), loss)


# speedup vs baseline: 3.3268x; 1.0003x over previous
import jax
import jax.numpy as jnp
from jax.experimental import pallas as pl
from jax.experimental.pallas import tpu as pltpu

_K = 5532
_C = 256
_KB = 512
_NBLK = 11
_NPIX = 2048
_N_TOT = 4096
_NPAD = _NBLK * _KB - _K


def _oim_kernel(lut_ref, x_ref, tgt_ref, out_ref, s_ref, tl_ref):
    b = pl.program_id(0)
    j = pl.program_id(1)

    @pl.when(j == 0)
    def _init():
        s_ref[...] = jnp.zeros((1, _NPIX), jnp.float32)
        tl_ref[...] = jnp.zeros((1, _NPIX), jnp.float32)

    x = x_ref[0]
    w = lut_ref[...]
    s_blk = jax.lax.dot_general(w, x, (((1,), (0,)), ((), ())),
                                preferred_element_type=jnp.float32)

    ones = jnp.ones((1, _KB), jnp.float32)
    p = jnp.exp(s_blk)
    s_ref[...] += jax.lax.dot_general(ones, p, (((1,), (0,)), ((), ())),
                                      preferred_element_type=jnp.float32)

    row = jax.lax.broadcasted_iota(jnp.int32, (_KB, _NPIX), 0) + j * _KB
    eq = row == tgt_ref[0]
    masked = jnp.where(eq, s_blk, 0.0)
    tl_ref[...] += jax.lax.dot_general(ones, masked, (((1,), (0,)), ((), ())),
                                       preferred_element_type=jnp.float32)

    @pl.when(j == _NBLK - 1)
    def _fin():
        nll = jnp.log(s_ref[...] - _NPAD) - tl_ref[...]
        part = jnp.sum(nll, axis=1, keepdims=True) * (1.0 / _N_TOT)

        @pl.when(b == 0)
        def _w():
            out_ref[...] = part

        @pl.when(b > 0)
        def _a():
            out_ref[...] += part


def kernel(lut, inputs, targets, epoch):
    lut_p = jnp.pad(lut.astype(jnp.bfloat16), ((0, _NBLK * _KB - _K), (0, 0)))
    x = inputs.reshape(2, _C, _NPIX).astype(jnp.bfloat16)
    tgt = targets.reshape(2, 1, _NPIX)
    out = pl.pallas_call(
        _oim_kernel,
        grid=(2, _NBLK),
        in_specs=[
            pl.BlockSpec((_KB, _C), lambda b, j: (j, 0)),
            pl.BlockSpec((1, _C, _NPIX), lambda b, j: (b, 0, 0)),
            pl.BlockSpec((1, 1, _NPIX), lambda b, j: (b, 0, 0)),
        ],
        out_specs=pl.BlockSpec((1, 1), lambda b, j: (0, 0)),
        out_shape=jax.ShapeDtypeStruct((1, 1), jnp.float32),
        scratch_shapes=[
            pltpu.VMEM((1, _NPIX), jnp.float32),
            pltpu.VMEM((1, _NPIX), jnp.float32),
        ],
        compiler_params=pltpu.CompilerParams(
            dimension_semantics=("arbitrary", "arbitrary"),
        ),
    )(lut_p, x, tgt)
    loss = out[0, 0]
    return jnp.where(epoch < 0, jnp.float32(0.0), loss)


# speedup vs baseline: 4.1599x; 1.2504x over previous
import jax
import jax.numpy as jnp
from jax.experimental import pallas as pl
from jax.experimental.pallas import tpu as pltpu

_K = 5532
_C = 256
_KB = 5632
_NBLK = 1
_NPIX = 2048
_N_TOT = 4096
_NPAD = _NBLK * _KB - _K


def _oim_kernel(lut_ref, x_ref, tgt_ref, out_ref, s_ref, tl_ref):
    b = pl.program_id(0)
    j = pl.program_id(1)

    @pl.when(j == 0)
    def _init():
        s_ref[...] = jnp.zeros((1, _NPIX), jnp.float32)
        tl_ref[...] = jnp.zeros((1, _NPIX), jnp.float32)

    x = x_ref[0]
    w = lut_ref[...]
    s_blk = jax.lax.dot_general(w, x, (((1,), (0,)), ((), ())),
                                preferred_element_type=jnp.float32)

    p = jnp.exp(s_blk)
    s_ref[...] += jnp.sum(p, axis=0, keepdims=True)

    row = jax.lax.broadcasted_iota(jnp.int32, (_KB, _NPIX), 0) + j * _KB
    eq = row == tgt_ref[0]
    masked = jnp.where(eq, s_blk, 0.0)
    tl_ref[...] += jnp.sum(masked, axis=0, keepdims=True)

    @pl.when(j == _NBLK - 1)
    def _fin():
        nll = jnp.log(s_ref[...] - _NPAD) - tl_ref[...]
        part = jnp.sum(nll, axis=1, keepdims=True) * (1.0 / _N_TOT)

        @pl.when(b == 0)
        def _w():
            out_ref[...] = part

        @pl.when(b > 0)
        def _a():
            out_ref[...] += part


def kernel(lut, inputs, targets, epoch):
    lut_p = jnp.pad(lut.astype(jnp.bfloat16), ((0, _NBLK * _KB - _K), (0, 0)))
    x = inputs.reshape(2, _C, _NPIX).astype(jnp.bfloat16)
    tgt = targets.reshape(2, 1, _NPIX)
    out = pl.pallas_call(
        _oim_kernel,
        grid=(2, _NBLK),
        in_specs=[
            pl.BlockSpec((_KB, _C), lambda b, j: (j, 0)),
            pl.BlockSpec((1, _C, _NPIX), lambda b, j: (b, 0, 0)),
            pl.BlockSpec((1, 1, _NPIX), lambda b, j: (b, 0, 0)),
        ],
        out_specs=pl.BlockSpec((1, 1), lambda b, j: (0, 0)),
        out_shape=jax.ShapeDtypeStruct((1, 1), jnp.float32),
        scratch_shapes=[
            pltpu.VMEM((1, _NPIX), jnp.float32),
            pltpu.VMEM((1, _NPIX), jnp.float32),
        ],
        compiler_params=pltpu.CompilerParams(
            dimension_semantics=("arbitrary", "arbitrary"),
        ),
    )(lut_p, x, tgt)
    loss = out[0, 0]
    return jnp.where(epoch < 0, jnp.float32(0.0), loss)


# speedup vs baseline: 4.4073x; 1.0595x over previous
import jax
import jax.numpy as jnp
from jax.experimental import pallas as pl
from jax.experimental.pallas import tpu as pltpu

_K = 5532
_C = 256
_NPIX = 2048
_N_TOT = 4096


def _oim_kernel(lut_ref, x_ref, tgt_ref, out_ref):
    b = pl.program_id(0)

    x = x_ref[0].astype(jnp.bfloat16)
    w = lut_ref[...].astype(jnp.bfloat16)
    s_blk = jax.lax.dot_general(w, x, (((1,), (0,)), ((), ())),
                                preferred_element_type=jnp.float32)

    p = jnp.exp(s_blk)
    s = jnp.sum(p, axis=0, keepdims=True)

    row = jax.lax.broadcasted_iota(jnp.int32, (_K, _NPIX), 0)
    eq = row == tgt_ref[0]
    tl = jnp.sum(jnp.where(eq, s_blk, 0.0), axis=0, keepdims=True)

    nll = jnp.log(s) - tl
    part = jnp.sum(nll, axis=1, keepdims=True) * (1.0 / _N_TOT)

    @pl.when(b == 0)
    def _w():
        out_ref[...] = part

    @pl.when(b > 0)
    def _a():
        out_ref[...] += part


def kernel(lut, inputs, targets, epoch):
    x = inputs.reshape(2, _C, _NPIX)
    tgt = targets.reshape(2, 1, _NPIX)
    out = pl.pallas_call(
        _oim_kernel,
        grid=(2,),
        in_specs=[
            pl.BlockSpec((_K, _C), lambda b: (0, 0)),
            pl.BlockSpec((1, _C, _NPIX), lambda b: (b, 0, 0)),
            pl.BlockSpec((1, 1, _NPIX), lambda b: (b, 0, 0)),
        ],
        out_specs=pl.BlockSpec((1, 1), lambda b: (0, 0)),
        out_shape=jax.ShapeDtypeStruct((1, 1), jnp.float32),
        compiler_params=pltpu.CompilerParams(
            dimension_semantics=("arbitrary",),
        ),
    )(lut, x, tgt)
    loss = out[0, 0]
    return jnp.where(epoch < 0, jnp.float32(0.0), loss)
